# Initial kernel scaffold; baseline (speedup 1.0000x reference)
import jax
import jax.numpy as jnp
from jax.experimental import pallas as pl


def kernel(x, edge_index, batch, hls_attr, edge_attr, params):
    raise NotImplementedError("write your pallas kernel here")


# speedup vs baseline: 4.7411x; 4.7411x over previous
import functools
import math

import jax
import jax.numpy as jnp
from jax import lax
from jax.experimental import pallas as pl
from jax.experimental.pallas import tpu as pltpu
from jax.experimental.pallas import tpu_sc as plsc

N = 10000
E = 320000
D = 128
H = 128
ED = 16
B = 64
L = 3

NB = 10
BLK = N // NB
HH = H // 2
ROWA = 96
ROWB = 64
QROW = 144
KVROW = 192

NCORES = 2
NSUB = 16
NTILES = NCORES * NSUB
EPT = E // NTILES
C = 80
NCHUNK = EPT // C
N_PAD = 10240
RPT = N_PAD // NSUB
ZR = 128

_INV_SQRT_H = 1.0 / math.sqrt(float(H))


def _proj_body(h_ref, w4_ref, b4_ref, wet_ref, qqe_ref, kv_ref, v2_ref, skip_ref):
    h = h_ref[...]
    out = jnp.dot(h, w4_ref[...], preferred_element_type=jnp.float32) + b4_ref[...]
    q = out[:, 0:H]
    qqe_ref[:, 0:H] = q
    qqe_ref[:, H:QROW] = jnp.dot(q, wet_ref[...], preferred_element_type=jnp.float32)
    kv_ref[...] = out[:, H:H + KVROW]
    v2_ref[...] = out[:, H + KVROW:3 * H]
    skip_ref[...] = out[:, 3 * H:]


def _proj(h, w4, b4, wet):
    return pl.pallas_call(
        _proj_body,
        grid=(NB,),
        in_specs=[
            pl.BlockSpec((BLK, D), lambda i: (i, 0)),
            pl.BlockSpec((D, 4 * H), lambda i: (0, 0)),
            pl.BlockSpec((1, 4 * H), lambda i: (0, 0)),
            pl.BlockSpec((H, ED), lambda i: (0, 0)),
        ],
        out_specs=[
            pl.BlockSpec((BLK, QROW), lambda i: (i, 0)),
            pl.BlockSpec((BLK, KVROW), lambda i: (i, 0)),
            pl.BlockSpec((BLK, HH), lambda i: (i, 0)),
            pl.BlockSpec((BLK, H), lambda i: (i, 0)),
        ],
        out_shape=[
            jax.ShapeDtypeStruct((N, QROW), jnp.float32),
            jax.ShapeDtypeStruct((N, KVROW), jnp.float32),
            jax.ShapeDtypeStruct((N, HH), jnp.float32),
            jax.ShapeDtypeStruct((N, H), jnp.float32),
        ],
    )(h, w4, b4, wet)


def _zero_acc(zb, acc, sid, row_w):
    zeros16 = jnp.zeros((16,), jnp.float32)

    def zrow(i, _):
        for j in range(row_w // 16):
            zb[i, pl.ds(j * 16, 16)] = zeros16
        return 0

    lax.fori_loop(0, ZR, zrow, 0)
    base_rows = sid * RPT
    for t in range(RPT // ZR):
        pltpu.sync_copy(zb, acc.at[pl.ds(base_rows + t * ZR, ZR)])


def _read_out(acc, out_hbm, cid, sid):
    base_rows = sid * RPT
    for t in range(RPT // ZR):
        r0 = base_rows + t * ZR
        pltpu.sync_copy(acc.at[pl.ds(r0, ZR)], out_hbm.at[cid, pl.ds(r0, ZR)])


def _edge_a_body(qqe_hbm, kv_hbm, ea_hbm, src_hbm, dst_hbm, out_hbm, eav_hbm,
                 srcb, dstb, qqeb, kvb, eab, bigb, eavb, zb, acc, sem):
    cid = lax.axis_index("c")
    sid = lax.axis_index("s")
    _zero_acc(zb, acc, sid, ROWA)
    plsc.subcore_barrier()

    ebase = (cid * NSUB + sid) * EPT

    def chunk(ci, _):
        eb = ebase + ci * C
        pltpu.sync_copy(src_hbm.at[pl.ds(eb, C)], srcb)
        pltpu.sync_copy(dst_hbm.at[pl.ds(eb, C)], dstb)
        pltpu.sync_copy(ea_hbm.at[pl.ds(eb, C)], eab)
        pltpu.async_copy(qqe_hbm.at[dstb], qqeb, sem).wait()
        pltpu.async_copy(kv_hbm.at[srcb], kvb, sem).wait()

        def edge(i, _):
            a16 = qqeb[i, pl.ds(H, 16)] * eab[i, :]
            for j in range(H // 16):
                a16 = a16 + qqeb[i, pl.ds(j * 16, 16)] * kvb[i, pl.ds(j * 16, 16)]
            tot = jnp.sum(a16) * _INV_SQRT_H
            eav = jnp.exp(jnp.full((16,), tot, jnp.float32))
            for j in range(HH // 16):
                bigb[i, pl.ds(j * 16, 16)] = kvb[i, pl.ds(H + j * 16, 16)] * eav
            bigb[i, pl.ds(HH, 16)] = eab[i, :] * eav
            bigb[i, pl.ds(HH + 16, 16)] = eav
            eavb[i, :] = eav
            return 0

        lax.fori_loop(0, C, edge, 0)
        pltpu.sync_copy(bigb, acc.at[dstb], add=True)
        pltpu.sync_copy(eavb, eav_hbm.at[pl.ds(eb, C)])
        return 0

    lax.fori_loop(0, NCHUNK, chunk, 0)
    plsc.subcore_barrier()
    _read_out(acc, out_hbm, cid, sid)


_SC_PARAMS = pltpu.CompilerParams(
    needs_layout_passes=False, use_tc_tiling_on_sc=False)


@functools.partial(
    pl.kernel,
    mesh=plsc.VectorSubcoreMesh(core_axis_name="c", subcore_axis_name="s"),
    out_type=(jax.ShapeDtypeStruct((NCORES, N_PAD, ROWA), jnp.float32),
              jax.ShapeDtypeStruct((E, 16), jnp.float32)),
    compiler_params=_SC_PARAMS,
    scratch_types=[
        pltpu.VMEM((C,), jnp.int32),
        pltpu.VMEM((C,), jnp.int32),
        pltpu.VMEM((C, QROW), jnp.float32),
        pltpu.VMEM((C, KVROW), jnp.float32),
        pltpu.VMEM((C, ED), jnp.float32),
        pltpu.VMEM((C, ROWA), jnp.float32),
        pltpu.VMEM((C, 16), jnp.float32),
        pltpu.VMEM((ZR, ROWA), jnp.float32),
        pltpu.VMEM_SHARED((N_PAD, ROWA), jnp.float32),
        pltpu.SemaphoreType.DMA,
    ],
)
def _edge_a(qqe_hbm, kv_hbm, ea_hbm, src_hbm, dst_hbm, out_hbm, eav_hbm,
            srcb, dstb, qqeb, kvb, eab, bigb, eavb, zb, acc, sem):
    _edge_a_body(qqe_hbm, kv_hbm, ea_hbm, src_hbm, dst_hbm, out_hbm, eav_hbm,
                 srcb, dstb, qqeb, kvb, eab, bigb, eavb, zb, acc, sem)


def _edge_b_body(v2_hbm, eav_hbm, src_hbm, dst_hbm, out_hbm,
                 srcb, dstb, v2b, bigb, eavb, zb, acc, sem):
    cid = lax.axis_index("c")
    sid = lax.axis_index("s")
    _zero_acc(zb, acc, sid, ROWB)
    plsc.subcore_barrier()

    ebase = (cid * NSUB + sid) * EPT

    def chunk(ci, _):
        eb = ebase + ci * C
        pltpu.sync_copy(src_hbm.at[pl.ds(eb, C)], srcb)
        pltpu.sync_copy(dst_hbm.at[pl.ds(eb, C)], dstb)
        pltpu.sync_copy(eav_hbm.at[pl.ds(eb, C)], eavb)
        pltpu.async_copy(v2_hbm.at[srcb], v2b, sem).wait()

        def edge(i, _):
            eav = eavb[i, :]
            for j in range(HH // 16):
                bigb[i, pl.ds(j * 16, 16)] = v2b[i, pl.ds(j * 16, 16)] * eav
            return 0

        lax.fori_loop(0, C, edge, 0)
        pltpu.sync_copy(bigb, acc.at[dstb], add=True)
        return 0

    lax.fori_loop(0, NCHUNK, chunk, 0)
    plsc.subcore_barrier()
    _read_out(acc, out_hbm, cid, sid)


@functools.partial(
    pl.kernel,
    mesh=plsc.VectorSubcoreMesh(core_axis_name="c", subcore_axis_name="s"),
    out_type=jax.ShapeDtypeStruct((NCORES, N_PAD, ROWB), jnp.float32),
    compiler_params=_SC_PARAMS,
    scratch_types=[
        pltpu.VMEM((C,), jnp.int32),
        pltpu.VMEM((C,), jnp.int32),
        pltpu.VMEM((C, HH), jnp.float32),
        pltpu.VMEM((C, ROWB), jnp.float32),
        pltpu.VMEM((C, 16), jnp.float32),
        pltpu.VMEM((ZR, ROWB), jnp.float32),
        pltpu.VMEM_SHARED((N_PAD, ROWB), jnp.float32),
        pltpu.SemaphoreType.DMA,
    ],
)
def _edge_b(v2_hbm, eav_hbm, src_hbm, dst_hbm, out_hbm,
            srcb, dstb, v2b, bigb, eavb, zb, acc, sem):
    _edge_b_body(v2_hbm, eav_hbm, src_hbm, dst_hbm, out_hbm,
                 srcb, dstb, v2b, bigb, eavb, zb, acc, sem)


def _finalize_body(pa_ref, pb_ref, skip_ref, we_ref, out_ref):
    pa = pa_ref[0] + pa_ref[1]
    pb = pb_ref[0] + pb_ref[1]
    num = jnp.concatenate([pa[:, 0:HH], pb], axis=1)
    a16 = pa[:, HH:HH + ED]
    den = pa[:, HH + ED:HH + ED + 1]
    seg = (num + jnp.dot(a16, we_ref[...], preferred_element_type=jnp.float32)) \
        / jnp.maximum(den, 1e-16)
    out_ref[...] = jnp.maximum(seg + skip_ref[...], 0.0)


def _finalize(pa, pb, skip, we):
    return pl.pallas_call(
        _finalize_body,
        grid=(NB,),
        in_specs=[
            pl.BlockSpec((NCORES, BLK, ROWA), lambda i: (0, i, 0)),
            pl.BlockSpec((NCORES, BLK, ROWB), lambda i: (0, i, 0)),
            pl.BlockSpec((BLK, H), lambda i: (i, 0)),
            pl.BlockSpec((ED, H), lambda i: (0, 0)),
        ],
        out_specs=pl.BlockSpec((BLK, H), lambda i: (i, 0)),
        out_shape=jax.ShapeDtypeStruct((N, H), jnp.float32),
    )(pa, pb, skip, we)


def _lstm_body(h1_ref, h2_ref, h3_ref, wihf_ref, whhf_ref, bf_ref,
               wihb_ref, whhb_ref, bb_ref, attw_ref, attb_ref, out_ref):
    xs = [h1_ref[...], h2_ref[...], h3_ref[...]]

    def cell(x, h, c, wih, whh, b):
        g = (jnp.dot(x, wih[...], preferred_element_type=jnp.float32)
             + jnp.dot(h, whh[...], preferred_element_type=jnp.float32) + b[...])
        gi = g[:, 0:H]
        gf = g[:, H:2 * H]
        gg = g[:, 2 * H:3 * H]
        go = g[:, 3 * H:4 * H]
        c2 = jax.nn.sigmoid(gf) * c + jax.nn.sigmoid(gi) * jnp.tanh(gg)
        h2 = jax.nn.sigmoid(go) * jnp.tanh(c2)
        return h2, c2

    z = jnp.zeros((BLK, H), jnp.float32)
    h, c = z, z
    hf = []
    for t in range(L):
        h, c = cell(xs[t], h, c, wihf_ref, whhf_ref, bf_ref)
        hf.append(h)
    h, c = z, z
    hb = [None] * L
    for t in range(L - 1, -1, -1):
        h, c = cell(xs[t], h, c, wihb_ref, whhb_ref, bb_ref)
        hb[t] = h

    attw = attw_ref[...]
    a = []
    for t in range(L):
        lo = jnp.concatenate([hf[t], hb[t]], axis=1)
        a.append(jnp.sum(lo * attw, axis=1, keepdims=True) + attb_ref[...])
    m = jnp.maximum(jnp.maximum(a[0], a[1]), a[2])
    e = [jnp.exp(x - m) for x in a]
    s = e[0] + e[1] + e[2]
    out_ref[...] = (xs[0] * e[0] + xs[1] * e[1] + xs[2] * e[2]) / s


def _lstm(hs, p):
    blk = lambda i: (i, 0)
    full = lambda i: (0, 0)
    return pl.pallas_call(
        _lstm_body,
        grid=(NB,),
        in_specs=[
            pl.BlockSpec((BLK, H), blk),
            pl.BlockSpec((BLK, H), blk),
            pl.BlockSpec((BLK, H), blk),
            pl.BlockSpec((H, 4 * H), full),
            pl.BlockSpec((H, 4 * H), full),
            pl.BlockSpec((1, 4 * H), full),
            pl.BlockSpec((H, 4 * H), full),
            pl.BlockSpec((H, 4 * H), full),
            pl.BlockSpec((1, 4 * H), full),
            pl.BlockSpec((1, 2 * H), full),
            pl.BlockSpec((1, 1), full),
        ],
        out_specs=pl.BlockSpec((BLK, H), blk),
        out_shape=jax.ShapeDtypeStruct((N, H), jnp.float32),
    )(hs[0], hs[1], hs[2],
      p['Wih_f'], p['Whh_f'], p['b_f'][None, :],
      p['Wih_b'], p['Whh_b'], p['b_b'][None, :],
      p['att_W'].T, p['att_b'][None, :])


def _pool_body(h_ref, batch_ref, hls_ref,
               w1p_ref, b1p_ref, w2p_ref, b2p_ref,
               w1t_ref, b1t_ref, w2t_ref, b2t_ref,
               mw0_ref, mb0_ref, mw1_ref, mb1_ref, mw2_ref, mb2_ref,
               out_ref, nump, denp, numt, dent):
    i = pl.program_id(0)
    h = h_ref[...]
    bt = batch_ref[...]
    a = (bt == lax.broadcasted_iota(jnp.int32, (BLK, B), 1)).astype(jnp.float32)

    def gate(w1_ref, b1_ref, w2_ref, b2_ref):
        g1 = jnp.maximum(
            jnp.dot(h, w1_ref[...], preferred_element_type=jnp.float32) + b1_ref[...], 0.0)
        g = jnp.dot(g1, w2_ref[...], preferred_element_type=jnp.float32) + b2_ref[...]
        return jnp.exp(g)

    egp = gate(w1p_ref, b1p_ref, w2p_ref, b2p_ref)
    egt = gate(w1t_ref, b1t_ref, w2t_ref, b2t_ref)

    dn = (((0,), (0,)), ((), ()))

    @pl.when(i == 0)
    def _():
        nump[...] = jnp.zeros_like(nump)
        denp[...] = jnp.zeros_like(denp)
        numt[...] = jnp.zeros_like(numt)
        dent[...] = jnp.zeros_like(dent)

    nump[...] += lax.dot_general(a, egp * h, dn, preferred_element_type=jnp.float32)
    denp[...] += lax.dot_general(a, egp, dn, preferred_element_type=jnp.float32)
    numt[...] += lax.dot_general(a, egt * h, dn, preferred_element_type=jnp.float32)
    dent[...] += lax.dot_general(a, egt, dn, preferred_element_type=jnp.float32)

    @pl.when(i == NB - 1)
    def _():
        outp = nump[...] / jnp.maximum(denp[...], 1e-16)
        outt = numt[...] / jnp.maximum(dent[...], 1e-16)
        zcat = jnp.concatenate([outp, outt, hls_ref[...]], axis=1)
        z1 = jnp.maximum(
            jnp.dot(zcat, mw0_ref[...], preferred_element_type=jnp.float32) + mb0_ref[...], 0.0)
        z2 = jnp.maximum(
            jnp.dot(z1, mw1_ref[...], preferred_element_type=jnp.float32) + mb1_ref[...], 0.0)
        out_ref[...] = jnp.dot(z2, mw2_ref[...], preferred_element_type=jnp.float32) + mb2_ref[...]


def _pool(h, batch2d, hls_attr, gp, gt, mlps):
    blk = lambda i: (i, 0)
    full = lambda i: (0, 0)
    return pl.pallas_call(
        _pool_body,
        grid=(NB,),
        in_specs=[
            pl.BlockSpec((BLK, H), blk),
            pl.BlockSpec((BLK, 1), blk),
            pl.BlockSpec((B, 64), full),
            pl.BlockSpec((H, H), full),
            pl.BlockSpec((1, H), full),
            pl.BlockSpec((H, 1), full),
            pl.BlockSpec((1, 1), full),
            pl.BlockSpec((H, H), full),
            pl.BlockSpec((1, H), full),
            pl.BlockSpec((H, 1), full),
            pl.BlockSpec((1, 1), full),
            pl.BlockSpec((2 * H + 64, 64), full),
            pl.BlockSpec((1, 64), full),
            pl.BlockSpec((64, 64), full),
            pl.BlockSpec((1, 64), full),
            pl.BlockSpec((64, 1), full),
            pl.BlockSpec((1, 1), full),
        ],
        out_specs=pl.BlockSpec((B, 1), full),
        out_shape=jax.ShapeDtypeStruct((B, 1), jnp.float32),
        scratch_shapes=[
            pltpu.VMEM((B, H), jnp.float32),
            pltpu.VMEM((B, 1), jnp.float32),
            pltpu.VMEM((B, H), jnp.float32),
            pltpu.VMEM((B, 1), jnp.float32),
        ],
    )(h, batch2d, hls_attr,
      gp['W1'], gp['b1'][None, :], gp['W2'], gp['b2'][None, :],
      gt['W1'], gt['b1'][None, :], gt['W2'], gt['b2'][None, :],
      mlps[0]['W'], mlps[0]['b'][None, :],
      mlps[1]['W'], mlps[1]['b'][None, :],
      mlps[2]['W'], mlps[2]['b'][None, :])


def kernel(x, edge_index, batch, hls_attr, edge_attr, params):
    src = edge_index[0]
    dst = edge_index[1]
    h = x
    hs = []
    for p in params['convs']:
        w4 = jnp.concatenate([p['Wq'], p['Wk'], p['Wv'], p['Wskip']], axis=1)
        b4 = jnp.concatenate([p['bq'], p['bk'], p['bv'], p['bskip']])[None, :]
        qqe, kv, v2, skip = _proj(h, w4, b4, p['We'].T)
        pa, eav = _edge_a(qqe, kv, edge_attr, src, dst)
        pb = _edge_b(v2, eav, src, dst)
        h = _finalize(pa[:, :N], pb[:, :N], skip, p['We'])
        hs.append(h)
    jk = _lstm(hs, params['lstm'])
    return _pool(jk, batch[:, None], hls_attr,
                 params['glob_P'], params['glob_T'], params['mlps'])


# speedup vs baseline: 5.8927x; 1.2429x over previous
import functools
import math

import jax
import jax.numpy as jnp
from jax import lax
from jax.experimental import pallas as pl
from jax.experimental.pallas import tpu as pltpu
from jax.experimental.pallas import tpu_sc as plsc

N = 10000
E = 320000
D = 128
H = 128
ED = 16
B = 64
L = 3

NB = 10
BLK = N // NB
HH = H // 2
ROWA = 96
ROWB = 64
QROW = 144
KVROW = 192

NCORES = 2
NSUB = 16
NTILES = NCORES * NSUB
EPT = E // NTILES
C = 40
NCHUNK = EPT // C
N_PAD = 10240
RPT = N_PAD // NSUB
ZR = 128

_INV_SQRT_H = 1.0 / math.sqrt(float(H))


def _proj_body(h_ref, w4_ref, b4_ref, wet_ref, qqe_ref, kv_ref, v2_ref, skip_ref):
    h = h_ref[...]
    out = jnp.dot(h, w4_ref[...], preferred_element_type=jnp.float32) + b4_ref[...]
    q = out[:, 0:H]
    qqe_ref[:, 0:H] = q
    qqe_ref[:, H:QROW] = jnp.dot(q, wet_ref[...], preferred_element_type=jnp.float32)
    kv_ref[...] = out[:, H:H + KVROW]
    v2_ref[...] = out[:, H + KVROW:3 * H]
    skip_ref[...] = out[:, 3 * H:]


def _proj(h, w4, b4, wet):
    return pl.pallas_call(
        _proj_body,
        grid=(NB,),
        in_specs=[
            pl.BlockSpec((BLK, D), lambda i: (i, 0)),
            pl.BlockSpec((D, 4 * H), lambda i: (0, 0)),
            pl.BlockSpec((1, 4 * H), lambda i: (0, 0)),
            pl.BlockSpec((H, ED), lambda i: (0, 0)),
        ],
        out_specs=[
            pl.BlockSpec((BLK, QROW), lambda i: (i, 0)),
            pl.BlockSpec((BLK, KVROW), lambda i: (i, 0)),
            pl.BlockSpec((BLK, HH), lambda i: (i, 0)),
            pl.BlockSpec((BLK, H), lambda i: (i, 0)),
        ],
        out_shape=[
            jax.ShapeDtypeStruct((N, QROW), jnp.float32),
            jax.ShapeDtypeStruct((N, KVROW), jnp.float32),
            jax.ShapeDtypeStruct((N, HH), jnp.float32),
            jax.ShapeDtypeStruct((N, H), jnp.float32),
        ],
    )(h, w4, b4, wet)


def _zero_acc(zb, acc, sid, row_w):
    zeros16 = jnp.zeros((16,), jnp.float32)

    def zrow(i, _):
        for j in range(row_w // 16):
            zb[i, pl.ds(j * 16, 16)] = zeros16
        return 0

    lax.fori_loop(0, ZR, zrow, 0)
    base_rows = sid * RPT
    for t in range(RPT // ZR):
        pltpu.sync_copy(zb, acc.at[pl.ds(base_rows + t * ZR, ZR)])


def _read_out(acc, out_hbm, cid, sid):
    base_rows = sid * RPT
    for t in range(RPT // ZR):
        r0 = base_rows + t * ZR
        pltpu.sync_copy(acc.at[pl.ds(r0, ZR)], out_hbm.at[cid, pl.ds(r0, ZR)])


def _edge_a_body(qqe_hbm, kv_hbm, ea_hbm, src_hbm, dst_hbm, out_hbm, eav_hbm,
                 srcb, dstb, qqeb, kvb, eab, bigb, eavb,
                 zb, acc, gsem):
    cid = lax.axis_index("c")
    sid = lax.axis_index("s")

    _zero_acc(zb, acc, sid, ROWA)
    plsc.subcore_barrier()

    ebase = (cid * NSUB + sid) * EPT

    def body(ci, _):
        eb = ebase + ci * C
        pltpu.sync_copy(src_hbm.at[pl.ds(eb, C)], srcb)
        pltpu.sync_copy(dst_hbm.at[pl.ds(eb, C)], dstb)
        pltpu.async_copy(ea_hbm.at[pl.ds(eb, C)], eab, gsem)
        pltpu.async_copy(qqe_hbm.at[dstb], qqeb, gsem)
        pltpu.async_copy(kv_hbm.at[srcb], kvb, gsem)
        pltpu.make_async_copy(ea_hbm.at[pl.ds(eb, C)], eab, gsem).wait()
        pltpu.make_async_copy(qqe_hbm.at[dstb], qqeb, gsem).wait()
        pltpu.make_async_copy(kv_hbm.at[srcb], kvb, gsem).wait()

        @plsc.parallel_loop(0, C)
        def edge(i):
            a16 = qqeb[i, pl.ds(H, 16)] * eab[i, :]
            for j in range(H // 16):
                a16 = a16 + qqeb[i, pl.ds(j * 16, 16)] * kvb[i, pl.ds(j * 16, 16)]
            tot = jnp.sum(a16) * _INV_SQRT_H
            eav = jnp.exp(jnp.full((16,), tot, jnp.float32))
            for j in range(HH // 16):
                bigb[i, pl.ds(j * 16, 16)] = kvb[i, pl.ds(H + j * 16, 16)] * eav
            bigb[i, pl.ds(HH, 16)] = eab[i, :] * eav
            bigb[i, pl.ds(HH + 16, 16)] = eav
            eavb[i, :] = eav

        pltpu.sync_copy(bigb, acc.at[dstb], add=True)
        pltpu.sync_copy(eavb, eav_hbm.at[pl.ds(eb, C)])
        return 0

    lax.fori_loop(0, NCHUNK, body, 0)

    plsc.subcore_barrier()
    _read_out(acc, out_hbm, cid, sid)


_SC_PARAMS = pltpu.CompilerParams(
    needs_layout_passes=False, use_tc_tiling_on_sc=False)


@functools.partial(
    pl.kernel,
    mesh=plsc.VectorSubcoreMesh(core_axis_name="c", subcore_axis_name="s"),
    out_type=(jax.ShapeDtypeStruct((NCORES, N_PAD, ROWA), jnp.float32),
              jax.ShapeDtypeStruct((E, 16), jnp.float32)),
    compiler_params=_SC_PARAMS,
    scratch_types=(
        [pltpu.VMEM((C,), jnp.int32),
         pltpu.VMEM((C,), jnp.int32),
         pltpu.VMEM((C, QROW), jnp.float32),
         pltpu.VMEM((C, KVROW), jnp.float32),
         pltpu.VMEM((C, ED), jnp.float32),
         pltpu.VMEM((C, ROWA), jnp.float32),
         pltpu.VMEM((C, 16), jnp.float32)]
        + [pltpu.VMEM((ZR, ROWA), jnp.float32),
           pltpu.VMEM_SHARED((N_PAD, ROWA), jnp.float32),
           pltpu.SemaphoreType.DMA]
    ),
)
def _edge_a(*refs):
    _edge_a_body(*refs)


def _edge_b_body(v2_hbm, eav_hbm, src_hbm, dst_hbm, out_hbm,
                 srcb, dstb, v2b, bigb, eavb,
                 zb, acc, gsem):
    cid = lax.axis_index("c")
    sid = lax.axis_index("s")

    _zero_acc(zb, acc, sid, ROWB)
    plsc.subcore_barrier()

    ebase = (cid * NSUB + sid) * EPT

    def body(ci, _):
        eb = ebase + ci * C
        pltpu.sync_copy(src_hbm.at[pl.ds(eb, C)], srcb)
        pltpu.sync_copy(dst_hbm.at[pl.ds(eb, C)], dstb)
        pltpu.async_copy(eav_hbm.at[pl.ds(eb, C)], eavb, gsem)
        pltpu.async_copy(v2_hbm.at[srcb], v2b, gsem)
        pltpu.make_async_copy(eav_hbm.at[pl.ds(eb, C)], eavb, gsem).wait()
        pltpu.make_async_copy(v2_hbm.at[srcb], v2b, gsem).wait()

        @plsc.parallel_loop(0, C)
        def edge(i):
            eav = eavb[i, :]
            for j in range(HH // 16):
                bigb[i, pl.ds(j * 16, 16)] = v2b[i, pl.ds(j * 16, 16)] * eav

        pltpu.sync_copy(bigb, acc.at[dstb], add=True)
        return 0

    lax.fori_loop(0, NCHUNK, body, 0)

    plsc.subcore_barrier()
    _read_out(acc, out_hbm, cid, sid)


@functools.partial(
    pl.kernel,
    mesh=plsc.VectorSubcoreMesh(core_axis_name="c", subcore_axis_name="s"),
    out_type=jax.ShapeDtypeStruct((NCORES, N_PAD, ROWB), jnp.float32),
    compiler_params=_SC_PARAMS,
    scratch_types=(
        [pltpu.VMEM((C,), jnp.int32),
         pltpu.VMEM((C,), jnp.int32),
         pltpu.VMEM((C, HH), jnp.float32),
         pltpu.VMEM((C, ROWB), jnp.float32),
         pltpu.VMEM((C, 16), jnp.float32)]
        + [pltpu.VMEM((ZR, ROWB), jnp.float32),
           pltpu.VMEM_SHARED((N_PAD, ROWB), jnp.float32),
           pltpu.SemaphoreType.DMA]
    ),
)
def _edge_b(*refs):
    _edge_b_body(*refs)


def _finalize_body(pa_ref, pb_ref, skip_ref, we_ref, out_ref):
    pa = pa_ref[0] + pa_ref[1]
    pb = pb_ref[0] + pb_ref[1]
    num = jnp.concatenate([pa[:, 0:HH], pb], axis=1)
    a16 = pa[:, HH:HH + ED]
    den = pa[:, HH + ED:HH + ED + 1]
    seg = (num + jnp.dot(a16, we_ref[...], preferred_element_type=jnp.float32)) \
        / jnp.maximum(den, 1e-16)
    out_ref[...] = jnp.maximum(seg + skip_ref[...], 0.0)


def _finalize(pa, pb, skip, we):
    return pl.pallas_call(
        _finalize_body,
        grid=(NB,),
        in_specs=[
            pl.BlockSpec((NCORES, BLK, ROWA), lambda i: (0, i, 0)),
            pl.BlockSpec((NCORES, BLK, ROWB), lambda i: (0, i, 0)),
            pl.BlockSpec((BLK, H), lambda i: (i, 0)),
            pl.BlockSpec((ED, H), lambda i: (0, 0)),
        ],
        out_specs=pl.BlockSpec((BLK, H), lambda i: (i, 0)),
        out_shape=jax.ShapeDtypeStruct((N, H), jnp.float32),
    )(pa, pb, skip, we)


def _lstm_body(h1_ref, h2_ref, h3_ref, wihf_ref, whhf_ref, bf_ref,
               wihb_ref, whhb_ref, bb_ref, attw_ref, attb_ref, out_ref):
    xs = [h1_ref[...], h2_ref[...], h3_ref[...]]

    def cell(x, h, c, wih, whh, b):
        g = (jnp.dot(x, wih[...], preferred_element_type=jnp.float32)
             + jnp.dot(h, whh[...], preferred_element_type=jnp.float32) + b[...])
        gi = g[:, 0:H]
        gf = g[:, H:2 * H]
        gg = g[:, 2 * H:3 * H]
        go = g[:, 3 * H:4 * H]
        c2 = jax.nn.sigmoid(gf) * c + jax.nn.sigmoid(gi) * jnp.tanh(gg)
        h2 = jax.nn.sigmoid(go) * jnp.tanh(c2)
        return h2, c2

    z = jnp.zeros((BLK, H), jnp.float32)
    h, c = z, z
    hf = []
    for t in range(L):
        h, c = cell(xs[t], h, c, wihf_ref, whhf_ref, bf_ref)
        hf.append(h)
    h, c = z, z
    hb = [None] * L
    for t in range(L - 1, -1, -1):
        h, c = cell(xs[t], h, c, wihb_ref, whhb_ref, bb_ref)
        hb[t] = h

    attw = attw_ref[...]
    a = []
    for t in range(L):
        lo = jnp.concatenate([hf[t], hb[t]], axis=1)
        a.append(jnp.sum(lo * attw, axis=1, keepdims=True) + attb_ref[...])
    m = jnp.maximum(jnp.maximum(a[0], a[1]), a[2])
    e = [jnp.exp(x - m) for x in a]
    s = e[0] + e[1] + e[2]
    out_ref[...] = (xs[0] * e[0] + xs[1] * e[1] + xs[2] * e[2]) / s


def _lstm(hs, p):
    blk = lambda i: (i, 0)
    full = lambda i: (0, 0)
    return pl.pallas_call(
        _lstm_body,
        grid=(NB,),
        in_specs=[
            pl.BlockSpec((BLK, H), blk),
            pl.BlockSpec((BLK, H), blk),
            pl.BlockSpec((BLK, H), blk),
            pl.BlockSpec((H, 4 * H), full),
            pl.BlockSpec((H, 4 * H), full),
            pl.BlockSpec((1, 4 * H), full),
            pl.BlockSpec((H, 4 * H), full),
            pl.BlockSpec((H, 4 * H), full),
            pl.BlockSpec((1, 4 * H), full),
            pl.BlockSpec((1, 2 * H), full),
            pl.BlockSpec((1, 1), full),
        ],
        out_specs=pl.BlockSpec((BLK, H), blk),
        out_shape=jax.ShapeDtypeStruct((N, H), jnp.float32),
    )(hs[0], hs[1], hs[2],
      p['Wih_f'], p['Whh_f'], p['b_f'][None, :],
      p['Wih_b'], p['Whh_b'], p['b_b'][None, :],
      p['att_W'].T, p['att_b'][None, :])


def _pool_body(h_ref, batch_ref, hls_ref,
               w1p_ref, b1p_ref, w2p_ref, b2p_ref,
               w1t_ref, b1t_ref, w2t_ref, b2t_ref,
               mw0_ref, mb0_ref, mw1_ref, mb1_ref, mw2_ref, mb2_ref,
               out_ref, nump, denp, numt, dent):
    i = pl.program_id(0)
    h = h_ref[...]
    bt = batch_ref[...]
    a = (bt == lax.broadcasted_iota(jnp.int32, (BLK, B), 1)).astype(jnp.float32)

    def gate(w1_ref, b1_ref, w2_ref, b2_ref):
        g1 = jnp.maximum(
            jnp.dot(h, w1_ref[...], preferred_element_type=jnp.float32) + b1_ref[...], 0.0)
        g = jnp.dot(g1, w2_ref[...], preferred_element_type=jnp.float32) + b2_ref[...]
        return jnp.exp(g)

    egp = gate(w1p_ref, b1p_ref, w2p_ref, b2p_ref)
    egt = gate(w1t_ref, b1t_ref, w2t_ref, b2t_ref)

    dn = (((0,), (0,)), ((), ()))

    @pl.when(i == 0)
    def _():
        nump[...] = jnp.zeros_like(nump)
        denp[...] = jnp.zeros_like(denp)
        numt[...] = jnp.zeros_like(numt)
        dent[...] = jnp.zeros_like(dent)

    nump[...] += lax.dot_general(a, egp * h, dn, preferred_element_type=jnp.float32)
    denp[...] += lax.dot_general(a, egp, dn, preferred_element_type=jnp.float32)
    numt[...] += lax.dot_general(a, egt * h, dn, preferred_element_type=jnp.float32)
    dent[...] += lax.dot_general(a, egt, dn, preferred_element_type=jnp.float32)

    @pl.when(i == NB - 1)
    def _():
        outp = nump[...] / jnp.maximum(denp[...], 1e-16)
        outt = numt[...] / jnp.maximum(dent[...], 1e-16)
        zcat = jnp.concatenate([outp, outt, hls_ref[...]], axis=1)
        z1 = jnp.maximum(
            jnp.dot(zcat, mw0_ref[...], preferred_element_type=jnp.float32) + mb0_ref[...], 0.0)
        z2 = jnp.maximum(
            jnp.dot(z1, mw1_ref[...], preferred_element_type=jnp.float32) + mb1_ref[...], 0.0)
        out_ref[...] = jnp.dot(z2, mw2_ref[...], preferred_element_type=jnp.float32) + mb2_ref[...]


def _pool(h, batch2d, hls_attr, gp, gt, mlps):
    blk = lambda i: (i, 0)
    full = lambda i: (0, 0)
    return pl.pallas_call(
        _pool_body,
        grid=(NB,),
        in_specs=[
            pl.BlockSpec((BLK, H), blk),
            pl.BlockSpec((BLK, 1), blk),
            pl.BlockSpec((B, 64), full),
            pl.BlockSpec((H, H), full),
            pl.BlockSpec((1, H), full),
            pl.BlockSpec((H, 1), full),
            pl.BlockSpec((1, 1), full),
            pl.BlockSpec((H, H), full),
            pl.BlockSpec((1, H), full),
            pl.BlockSpec((H, 1), full),
            pl.BlockSpec((1, 1), full),
            pl.BlockSpec((2 * H + 64, 64), full),
            pl.BlockSpec((1, 64), full),
            pl.BlockSpec((64, 64), full),
            pl.BlockSpec((1, 64), full),
            pl.BlockSpec((64, 1), full),
            pl.BlockSpec((1, 1), full),
        ],
        out_specs=pl.BlockSpec((B, 1), full),
        out_shape=jax.ShapeDtypeStruct((B, 1), jnp.float32),
        scratch_shapes=[
            pltpu.VMEM((B, H), jnp.float32),
            pltpu.VMEM((B, 1), jnp.float32),
            pltpu.VMEM((B, H), jnp.float32),
            pltpu.VMEM((B, 1), jnp.float32),
        ],
    )(h, batch2d, hls_attr,
      gp['W1'], gp['b1'][None, :], gp['W2'], gp['b2'][None, :],
      gt['W1'], gt['b1'][None, :], gt['W2'], gt['b2'][None, :],
      mlps[0]['W'], mlps[0]['b'][None, :],
      mlps[1]['W'], mlps[1]['b'][None, :],
      mlps[2]['W'], mlps[2]['b'][None, :])


def kernel(x, edge_index, batch, hls_attr, edge_attr, params):
    src = edge_index[0]
    dst = edge_index[1]
    h = x
    hs = []
    for p in params['convs']:
        w4 = jnp.concatenate([p['Wq'], p['Wk'], p['Wv'], p['Wskip']], axis=1)
        b4 = jnp.concatenate([p['bq'], p['bk'], p['bv'], p['bskip']])[None, :]
        qqe, kv, v2, skip = _proj(h, w4, b4, p['We'].T)
        pa, eav = _edge_a(qqe, kv, edge_attr, src, dst)
        pb = _edge_b(v2, eav, src, dst)
        h = _finalize(pa[:, :N], pb[:, :N], skip, p['We'])
        hs.append(h)
    jk = _lstm(hs, params['lstm'])
    return _pool(jk, batch[:, None], hls_attr,
                 params['glob_P'], params['glob_T'], params['mlps'])


# speedup vs baseline: 8.2182x; 1.3946x over previous
import functools
import math

import jax
import jax.numpy as jnp
from jax import lax
from jax.experimental import pallas as pl
from jax.experimental.pallas import tpu as pltpu
from jax.experimental.pallas import tpu_sc as plsc

N = 10000
E = 320000
D = 128
H = 128
ED = 16
B = 64
L = 3

NB = 10
BLK = N // NB
ROW = 160
QROW = 144
KVROW = 256

NCORES = 2
NSUB = 16
NTILES = NCORES * NSUB
EPT = E // NTILES
C = 40
NCHUNK = EPT // C
N_PAD = 10240
RPT = N_PAD // NSUB
ZR = 16

_INV_SQRT_H = 1.0 / math.sqrt(float(H))


def _proj_body(h_ref, w4_ref, b4_ref, wet_ref, qqe_ref, kv_ref, skip_ref):
    h = h_ref[...]
    out = jnp.dot(h, w4_ref[...], preferred_element_type=jnp.float32) + b4_ref[...]
    q = out[:, 0:H]
    qqe_ref[:, 0:H] = q
    qqe_ref[:, H:QROW] = jnp.dot(q, wet_ref[...], preferred_element_type=jnp.float32)
    kv_ref[...] = out[:, H:H + KVROW]
    skip_ref[...] = out[:, 3 * H:]


def _proj(h, w4, b4, wet):
    return pl.pallas_call(
        _proj_body,
        grid=(NB,),
        in_specs=[
            pl.BlockSpec((BLK, D), lambda i: (i, 0)),
            pl.BlockSpec((D, 4 * H), lambda i: (0, 0)),
            pl.BlockSpec((1, 4 * H), lambda i: (0, 0)),
            pl.BlockSpec((H, ED), lambda i: (0, 0)),
        ],
        out_specs=[
            pl.BlockSpec((BLK, QROW), lambda i: (i, 0)),
            pl.BlockSpec((BLK, KVROW), lambda i: (i, 0)),
            pl.BlockSpec((BLK, H), lambda i: (i, 0)),
        ],
        out_shape=[
            jax.ShapeDtypeStruct((N, QROW), jnp.float32),
            jax.ShapeDtypeStruct((N, KVROW), jnp.float32),
            jax.ShapeDtypeStruct((N, H), jnp.float32),
        ],
    )(h, w4, b4, wet)


def _zero_acc(zb, acc, sid, row_w):
    zeros16 = jnp.zeros((16,), jnp.float32)

    def zrow(i, _):
        for j in range(row_w // 16):
            zb[i, pl.ds(j * 16, 16)] = zeros16
        return 0

    lax.fori_loop(0, ZR, zrow, 0)
    base_rows = sid * RPT
    for t in range(RPT // ZR):
        pltpu.sync_copy(zb, acc.at[pl.ds(base_rows + t * ZR, ZR)])


def _read_out(acc, out_hbm, cid, sid):
    base_rows = sid * RPT
    for t in range(RPT // ZR):
        r0 = base_rows + t * ZR
        pltpu.sync_copy(acc.at[pl.ds(r0, ZR)], out_hbm.at[cid, pl.ds(r0, ZR)])


def _edge_body(qqe_hbm, kv_hbm, ea_hbm, src_hbm, dst_hbm, out_hbm,
               srcb, dstb, qqeb, kvb, eab, bigb,
               zb, acc, gsem):
    cid = lax.axis_index("c")
    sid = lax.axis_index("s")

    _zero_acc(zb, acc, sid, ROW)
    plsc.subcore_barrier()

    ebase = (cid * NSUB + sid) * EPT

    def body(ci, _):
        eb = ebase + ci * C
        pltpu.sync_copy(src_hbm.at[pl.ds(eb, C)], srcb)
        pltpu.sync_copy(dst_hbm.at[pl.ds(eb, C)], dstb)
        pltpu.async_copy(ea_hbm.at[pl.ds(eb, C)], eab, gsem)
        pltpu.async_copy(qqe_hbm.at[dstb], qqeb, gsem)
        pltpu.async_copy(kv_hbm.at[srcb], kvb, gsem)
        pltpu.make_async_copy(ea_hbm.at[pl.ds(eb, C)], eab, gsem).wait()
        pltpu.make_async_copy(qqe_hbm.at[dstb], qqeb, gsem).wait()
        pltpu.make_async_copy(kv_hbm.at[srcb], kvb, gsem).wait()

        @plsc.parallel_loop(0, C)
        def edge(i):
            a16 = qqeb[i, pl.ds(H, 16)] * eab[i, :]
            for j in range(H // 16):
                a16 = a16 + qqeb[i, pl.ds(j * 16, 16)] * kvb[i, pl.ds(j * 16, 16)]
            tot = jnp.sum(a16) * _INV_SQRT_H
            eav = jnp.exp(jnp.full((16,), tot, jnp.float32))
            for j in range(H // 16):
                bigb[i, pl.ds(j * 16, 16)] = kvb[i, pl.ds(H + j * 16, 16)] * eav
            bigb[i, pl.ds(H, 16)] = eab[i, :] * eav
            bigb[i, pl.ds(H + 16, 16)] = eav

        pltpu.sync_copy(bigb, acc.at[dstb], add=True)
        return 0

    lax.fori_loop(0, NCHUNK, body, 0)

    plsc.subcore_barrier()
    _read_out(acc, out_hbm, cid, sid)


_SC_PARAMS = pltpu.CompilerParams(
    needs_layout_passes=False, use_tc_tiling_on_sc=False)


@functools.partial(
    pl.kernel,
    mesh=plsc.VectorSubcoreMesh(core_axis_name="c", subcore_axis_name="s"),
    out_type=jax.ShapeDtypeStruct((NCORES, N_PAD, ROW), jnp.float32),
    compiler_params=_SC_PARAMS,
    scratch_types=(
        [pltpu.VMEM((C,), jnp.int32),
         pltpu.VMEM((C,), jnp.int32),
         pltpu.VMEM((C, QROW), jnp.float32),
         pltpu.VMEM((C, KVROW), jnp.float32),
         pltpu.VMEM((C, ED), jnp.float32),
         pltpu.VMEM((C, ROW), jnp.float32)]
        + [pltpu.VMEM((ZR, ROW), jnp.float32),
           pltpu.VMEM_SHARED((N_PAD, ROW), jnp.float32),
           pltpu.SemaphoreType.DMA]
    ),
)
def _edge(*refs):
    _edge_body(*refs)


def _finalize_body(pa_ref, skip_ref, we_ref, out_ref):
    pa = pa_ref[0] + pa_ref[1]
    num = pa[:, 0:H]
    a16 = pa[:, H:H + ED]
    den = pa[:, H + ED:H + ED + 1]
    seg = (num + jnp.dot(a16, we_ref[...], preferred_element_type=jnp.float32)) \
        / jnp.maximum(den, 1e-16)
    out_ref[...] = jnp.maximum(seg + skip_ref[...], 0.0)


def _finalize(pa, skip, we):
    return pl.pallas_call(
        _finalize_body,
        grid=(NB,),
        in_specs=[
            pl.BlockSpec((NCORES, BLK, ROW), lambda i: (0, i, 0)),
            pl.BlockSpec((BLK, H), lambda i: (i, 0)),
            pl.BlockSpec((ED, H), lambda i: (0, 0)),
        ],
        out_specs=pl.BlockSpec((BLK, H), lambda i: (i, 0)),
        out_shape=jax.ShapeDtypeStruct((N, H), jnp.float32),
    )(pa, skip, we)


def _lstm_body(h1_ref, h2_ref, h3_ref, wihf_ref, whhf_ref, bf_ref,
               wihb_ref, whhb_ref, bb_ref, attw_ref, attb_ref, out_ref):
    xs = [h1_ref[...], h2_ref[...], h3_ref[...]]

    def cell(x, h, c, wih, whh, b):
        g = (jnp.dot(x, wih[...], preferred_element_type=jnp.float32)
             + jnp.dot(h, whh[...], preferred_element_type=jnp.float32) + b[...])
        gi = g[:, 0:H]
        gf = g[:, H:2 * H]
        gg = g[:, 2 * H:3 * H]
        go = g[:, 3 * H:4 * H]
        c2 = jax.nn.sigmoid(gf) * c + jax.nn.sigmoid(gi) * jnp.tanh(gg)
        h2 = jax.nn.sigmoid(go) * jnp.tanh(c2)
        return h2, c2

    z = jnp.zeros((BLK, H), jnp.float32)
    h, c = z, z
    hf = []
    for t in range(L):
        h, c = cell(xs[t], h, c, wihf_ref, whhf_ref, bf_ref)
        hf.append(h)
    h, c = z, z
    hb = [None] * L
    for t in range(L - 1, -1, -1):
        h, c = cell(xs[t], h, c, wihb_ref, whhb_ref, bb_ref)
        hb[t] = h

    attw = attw_ref[...]
    a = []
    for t in range(L):
        lo = jnp.concatenate([hf[t], hb[t]], axis=1)
        a.append(jnp.sum(lo * attw, axis=1, keepdims=True) + attb_ref[...])
    m = jnp.maximum(jnp.maximum(a[0], a[1]), a[2])
    e = [jnp.exp(x - m) for x in a]
    s = e[0] + e[1] + e[2]
    out_ref[...] = (xs[0] * e[0] + xs[1] * e[1] + xs[2] * e[2]) / s


def _lstm(hs, p):
    blk = lambda i: (i, 0)
    full = lambda i: (0, 0)
    return pl.pallas_call(
        _lstm_body,
        grid=(NB,),
        in_specs=[
            pl.BlockSpec((BLK, H), blk),
            pl.BlockSpec((BLK, H), blk),
            pl.BlockSpec((BLK, H), blk),
            pl.BlockSpec((H, 4 * H), full),
            pl.BlockSpec((H, 4 * H), full),
            pl.BlockSpec((1, 4 * H), full),
            pl.BlockSpec((H, 4 * H), full),
            pl.BlockSpec((H, 4 * H), full),
            pl.BlockSpec((1, 4 * H), full),
            pl.BlockSpec((1, 2 * H), full),
            pl.BlockSpec((1, 1), full),
        ],
        out_specs=pl.BlockSpec((BLK, H), blk),
        out_shape=jax.ShapeDtypeStruct((N, H), jnp.float32),
    )(hs[0], hs[1], hs[2],
      p['Wih_f'], p['Whh_f'], p['b_f'][None, :],
      p['Wih_b'], p['Whh_b'], p['b_b'][None, :],
      p['att_W'].T, p['att_b'][None, :])


def _pool_body(h_ref, batch_ref, hls_ref,
               w1p_ref, b1p_ref, w2p_ref, b2p_ref,
               w1t_ref, b1t_ref, w2t_ref, b2t_ref,
               mw0_ref, mb0_ref, mw1_ref, mb1_ref, mw2_ref, mb2_ref,
               out_ref, nump, denp, numt, dent):
    i = pl.program_id(0)
    h = h_ref[...]
    bt = batch_ref[...]
    a = (bt == lax.broadcasted_iota(jnp.int32, (BLK, B), 1)).astype(jnp.float32)

    def gate(w1_ref, b1_ref, w2_ref, b2_ref):
        g1 = jnp.maximum(
            jnp.dot(h, w1_ref[...], preferred_element_type=jnp.float32) + b1_ref[...], 0.0)
        g = jnp.dot(g1, w2_ref[...], preferred_element_type=jnp.float32) + b2_ref[...]
        return jnp.exp(g)

    egp = gate(w1p_ref, b1p_ref, w2p_ref, b2p_ref)
    egt = gate(w1t_ref, b1t_ref, w2t_ref, b2t_ref)

    dn = (((0,), (0,)), ((), ()))

    @pl.when(i == 0)
    def _():
        nump[...] = jnp.zeros_like(nump)
        denp[...] = jnp.zeros_like(denp)
        numt[...] = jnp.zeros_like(numt)
        dent[...] = jnp.zeros_like(dent)

    nump[...] += lax.dot_general(a, egp * h, dn, preferred_element_type=jnp.float32)
    denp[...] += lax.dot_general(a, egp, dn, preferred_element_type=jnp.float32)
    numt[...] += lax.dot_general(a, egt * h, dn, preferred_element_type=jnp.float32)
    dent[...] += lax.dot_general(a, egt, dn, preferred_element_type=jnp.float32)

    @pl.when(i == NB - 1)
    def _():
        outp = nump[...] / jnp.maximum(denp[...], 1e-16)
        outt = numt[...] / jnp.maximum(dent[...], 1e-16)
        zcat = jnp.concatenate([outp, outt, hls_ref[...]], axis=1)
        z1 = jnp.maximum(
            jnp.dot(zcat, mw0_ref[...], preferred_element_type=jnp.float32) + mb0_ref[...], 0.0)
        z2 = jnp.maximum(
            jnp.dot(z1, mw1_ref[...], preferred_element_type=jnp.float32) + mb1_ref[...], 0.0)
        out_ref[...] = jnp.dot(z2, mw2_ref[...], preferred_element_type=jnp.float32) + mb2_ref[...]


def _pool(h, batch2d, hls_attr, gp, gt, mlps):
    blk = lambda i: (i, 0)
    full = lambda i: (0, 0)
    return pl.pallas_call(
        _pool_body,
        grid=(NB,),
        in_specs=[
            pl.BlockSpec((BLK, H), blk),
            pl.BlockSpec((BLK, 1), blk),
            pl.BlockSpec((B, 64), full),
            pl.BlockSpec((H, H), full),
            pl.BlockSpec((1, H), full),
            pl.BlockSpec((H, 1), full),
            pl.BlockSpec((1, 1), full),
            pl.BlockSpec((H, H), full),
            pl.BlockSpec((1, H), full),
            pl.BlockSpec((H, 1), full),
            pl.BlockSpec((1, 1), full),
            pl.BlockSpec((2 * H + 64, 64), full),
            pl.BlockSpec((1, 64), full),
            pl.BlockSpec((64, 64), full),
            pl.BlockSpec((1, 64), full),
            pl.BlockSpec((64, 1), full),
            pl.BlockSpec((1, 1), full),
        ],
        out_specs=pl.BlockSpec((B, 1), full),
        out_shape=jax.ShapeDtypeStruct((B, 1), jnp.float32),
        scratch_shapes=[
            pltpu.VMEM((B, H), jnp.float32),
            pltpu.VMEM((B, 1), jnp.float32),
            pltpu.VMEM((B, H), jnp.float32),
            pltpu.VMEM((B, 1), jnp.float32),
        ],
    )(h, batch2d, hls_attr,
      gp['W1'], gp['b1'][None, :], gp['W2'], gp['b2'][None, :],
      gt['W1'], gt['b1'][None, :], gt['W2'], gt['b2'][None, :],
      mlps[0]['W'], mlps[0]['b'][None, :],
      mlps[1]['W'], mlps[1]['b'][None, :],
      mlps[2]['W'], mlps[2]['b'][None, :])


def kernel(x, edge_index, batch, hls_attr, edge_attr, params):
    src = edge_index[0]
    dst = edge_index[1]
    h = x
    hs = []
    for p in params['convs']:
        w4 = jnp.concatenate([p['Wq'], p['Wk'], p['Wv'], p['Wskip']], axis=1)
        b4 = jnp.concatenate([p['bq'], p['bk'], p['bv'], p['bskip']])[None, :]
        qqe, kv, skip = _proj(h, w4, b4, p['We'].T)
        pa = _edge(qqe, kv, edge_attr, src, dst)
        h = _finalize(pa[:, :N], skip, p['We'])
        hs.append(h)
    jk = _lstm(hs, params['lstm'])
    return _pool(jk, batch[:, None], hls_attr,
                 params['glob_P'], params['glob_T'], params['mlps'])


# speedup vs baseline: 8.8722x; 1.0796x over previous
import functools
import math

import jax
import jax.numpy as jnp
from jax import lax
from jax.experimental import pallas as pl
from jax.experimental.pallas import tpu as pltpu
from jax.experimental.pallas import tpu_sc as plsc

N = 10000
E = 320000
D = 128
H = 128
ED = 16
B = 64
L = 3

NB = 10
BLK = N // NB
ROW = 160
QROW = 144
KVROW = 256

NCORES = 2
NSUB = 16
NTILES = NCORES * NSUB
EPT = E // NTILES
C = 40
NCHUNK = EPT // C
N_PAD = 10240
RPT = N_PAD // NSUB
ZR = 16

_INV_SQRT_H = 1.0 / math.sqrt(float(H))


def _proj_body(h_ref, w4_ref, b4_ref, wet_ref, qqe_ref, kv_ref, skip_ref):
    h = h_ref[...]
    out = jnp.dot(h, w4_ref[...], preferred_element_type=jnp.float32) + b4_ref[...]
    q = out[:, 0:H]
    qqe_ref[:, 0:H] = q
    qqe_ref[:, H:QROW] = jnp.dot(q, wet_ref[...], preferred_element_type=jnp.float32)
    kv_ref[...] = out[:, H:H + KVROW]
    skip_ref[...] = out[:, 3 * H:]


def _proj(h, w4, b4, wet):
    return pl.pallas_call(
        _proj_body,
        grid=(NB,),
        in_specs=[
            pl.BlockSpec((BLK, D), lambda i: (i, 0)),
            pl.BlockSpec((D, 4 * H), lambda i: (0, 0)),
            pl.BlockSpec((1, 4 * H), lambda i: (0, 0)),
            pl.BlockSpec((H, ED), lambda i: (0, 0)),
        ],
        out_specs=[
            pl.BlockSpec((BLK, QROW), lambda i: (i, 0)),
            pl.BlockSpec((BLK, KVROW), lambda i: (i, 0)),
            pl.BlockSpec((BLK, H), lambda i: (i, 0)),
        ],
        out_shape=[
            jax.ShapeDtypeStruct((N, QROW), jnp.float32),
            jax.ShapeDtypeStruct((N, KVROW), jnp.float32),
            jax.ShapeDtypeStruct((N, H), jnp.float32),
        ],
    )(h, w4, b4, wet)


def _zero_acc(zb, acc, sid, row_w):
    zeros16 = jnp.zeros((16,), jnp.float32)

    def zrow(i, _):
        for j in range(row_w // 16):
            zb[i, pl.ds(j * 16, 16)] = zeros16
        return 0

    lax.fori_loop(0, ZR, zrow, 0)
    base_rows = sid * RPT
    for t in range(RPT // ZR):
        pltpu.sync_copy(zb, acc.at[pl.ds(base_rows + t * ZR, ZR)])


def _read_out(acc, out_hbm, cid, sid):
    base_rows = sid * RPT
    for t in range(RPT // ZR):
        r0 = base_rows + t * ZR
        pltpu.sync_copy(acc.at[pl.ds(r0, ZR)], out_hbm.at[cid, pl.ds(r0, ZR)])


def _edge_body(qqe_hbm, kv_hbm, ea_hbm, src_hbm, dst_hbm, out_hbm,
               srcb, dstb0, dstb1, qqeb, kvb, eab, bigb,
               zb, acc, gsem):
    cid = lax.axis_index("c")
    sid = lax.axis_index("s")

    _zero_acc(zb, acc, sid, ROW)
    plsc.subcore_barrier()

    ebase = (cid * NSUB + sid) * EPT

    def load_issue(eb, dstb):
        pltpu.sync_copy(src_hbm.at[pl.ds(eb, C)], srcb)
        pltpu.sync_copy(dst_hbm.at[pl.ds(eb, C)], dstb)
        pltpu.async_copy(ea_hbm.at[pl.ds(eb, C)], eab, gsem)
        pltpu.async_copy(qqe_hbm.at[dstb], qqeb, gsem)
        pltpu.async_copy(kv_hbm.at[srcb], kvb, gsem)

    def wait_compute(eb, dstb):
        pltpu.make_async_copy(ea_hbm.at[pl.ds(eb, C)], eab, gsem).wait()
        pltpu.make_async_copy(qqe_hbm.at[dstb], qqeb, gsem).wait()
        pltpu.make_async_copy(kv_hbm.at[srcb], kvb, gsem).wait()

        @plsc.parallel_loop(0, C)
        def edge(i):
            a16 = qqeb[i, pl.ds(H, 16)] * eab[i, :]
            for j in range(H // 16):
                a16 = a16 + qqeb[i, pl.ds(j * 16, 16)] * kvb[i, pl.ds(j * 16, 16)]
            tot = jnp.sum(a16) * _INV_SQRT_H
            eav = jnp.exp(jnp.full((16,), tot, jnp.float32))
            for j in range(H // 16):
                bigb[i, pl.ds(j * 16, 16)] = kvb[i, pl.ds(H + j * 16, 16)] * eav
            bigb[i, pl.ds(H, 16)] = eab[i, :] * eav
            bigb[i, pl.ds(H + 16, 16)] = eav

    load_issue(ebase, dstb0)

    def pair(t, _):
        ea_ = ebase + 2 * t * C
        wait_compute(ea_, dstb0)
        load_issue(ea_ + C, dstb1)
        pltpu.sync_copy(bigb, acc.at[dstb0], add=True)
        wait_compute(ea_ + C, dstb1)
        load_issue(ea_ + 2 * C, dstb0)
        pltpu.sync_copy(bigb, acc.at[dstb1], add=True)
        return 0

    lax.fori_loop(0, NCHUNK // 2 - 1, pair, 0)
    elast = ebase + (NCHUNK - 2) * C
    wait_compute(elast, dstb0)
    load_issue(elast + C, dstb1)
    pltpu.sync_copy(bigb, acc.at[dstb0], add=True)
    wait_compute(elast + C, dstb1)
    pltpu.sync_copy(bigb, acc.at[dstb1], add=True)

    plsc.subcore_barrier()
    _read_out(acc, out_hbm, cid, sid)


_SC_PARAMS = pltpu.CompilerParams(
    needs_layout_passes=False, use_tc_tiling_on_sc=False)


@functools.partial(
    pl.kernel,
    mesh=plsc.VectorSubcoreMesh(core_axis_name="c", subcore_axis_name="s"),
    out_type=jax.ShapeDtypeStruct((NCORES, N_PAD, ROW), jnp.float32),
    compiler_params=_SC_PARAMS,
    scratch_types=(
        [pltpu.VMEM((C,), jnp.int32),
         pltpu.VMEM((C,), jnp.int32),
         pltpu.VMEM((C,), jnp.int32),
         pltpu.VMEM((C, QROW), jnp.float32),
         pltpu.VMEM((C, KVROW), jnp.float32),
         pltpu.VMEM((C, ED), jnp.float32),
         pltpu.VMEM((C, ROW), jnp.float32)]
        + [pltpu.VMEM((ZR, ROW), jnp.float32),
           pltpu.VMEM_SHARED((N_PAD, ROW), jnp.float32),
           pltpu.SemaphoreType.DMA]
    ),
)
def _edge(*refs):
    _edge_body(*refs)


def _finalize_body(pa_ref, skip_ref, we_ref, out_ref):
    pa = pa_ref[0] + pa_ref[1]
    num = pa[:, 0:H]
    a16 = pa[:, H:H + ED]
    den = pa[:, H + ED:H + ED + 1]
    seg = (num + jnp.dot(a16, we_ref[...], preferred_element_type=jnp.float32)) \
        / jnp.maximum(den, 1e-16)
    out_ref[...] = jnp.maximum(seg + skip_ref[...], 0.0)


def _finalize(pa, skip, we):
    return pl.pallas_call(
        _finalize_body,
        grid=(NB,),
        in_specs=[
            pl.BlockSpec((NCORES, BLK, ROW), lambda i: (0, i, 0)),
            pl.BlockSpec((BLK, H), lambda i: (i, 0)),
            pl.BlockSpec((ED, H), lambda i: (0, 0)),
        ],
        out_specs=pl.BlockSpec((BLK, H), lambda i: (i, 0)),
        out_shape=jax.ShapeDtypeStruct((N, H), jnp.float32),
    )(pa, skip, we)


def _lstm_body(h1_ref, h2_ref, h3_ref, wihf_ref, whhf_ref, bf_ref,
               wihb_ref, whhb_ref, bb_ref, attw_ref, attb_ref, out_ref):
    xs = [h1_ref[...], h2_ref[...], h3_ref[...]]

    def cell(x, h, c, wih, whh, b):
        g = (jnp.dot(x, wih[...], preferred_element_type=jnp.float32)
             + jnp.dot(h, whh[...], preferred_element_type=jnp.float32) + b[...])
        gi = g[:, 0:H]
        gf = g[:, H:2 * H]
        gg = g[:, 2 * H:3 * H]
        go = g[:, 3 * H:4 * H]
        c2 = jax.nn.sigmoid(gf) * c + jax.nn.sigmoid(gi) * jnp.tanh(gg)
        h2 = jax.nn.sigmoid(go) * jnp.tanh(c2)
        return h2, c2

    z = jnp.zeros((BLK, H), jnp.float32)
    h, c = z, z
    hf = []
    for t in range(L):
        h, c = cell(xs[t], h, c, wihf_ref, whhf_ref, bf_ref)
        hf.append(h)
    h, c = z, z
    hb = [None] * L
    for t in range(L - 1, -1, -1):
        h, c = cell(xs[t], h, c, wihb_ref, whhb_ref, bb_ref)
        hb[t] = h

    attw = attw_ref[...]
    a = []
    for t in range(L):
        lo = jnp.concatenate([hf[t], hb[t]], axis=1)
        a.append(jnp.sum(lo * attw, axis=1, keepdims=True) + attb_ref[...])
    m = jnp.maximum(jnp.maximum(a[0], a[1]), a[2])
    e = [jnp.exp(x - m) for x in a]
    s = e[0] + e[1] + e[2]
    out_ref[...] = (xs[0] * e[0] + xs[1] * e[1] + xs[2] * e[2]) / s


def _lstm(hs, p):
    blk = lambda i: (i, 0)
    full = lambda i: (0, 0)
    return pl.pallas_call(
        _lstm_body,
        grid=(NB,),
        in_specs=[
            pl.BlockSpec((BLK, H), blk),
            pl.BlockSpec((BLK, H), blk),
            pl.BlockSpec((BLK, H), blk),
            pl.BlockSpec((H, 4 * H), full),
            pl.BlockSpec((H, 4 * H), full),
            pl.BlockSpec((1, 4 * H), full),
            pl.BlockSpec((H, 4 * H), full),
            pl.BlockSpec((H, 4 * H), full),
            pl.BlockSpec((1, 4 * H), full),
            pl.BlockSpec((1, 2 * H), full),
            pl.BlockSpec((1, 1), full),
        ],
        out_specs=pl.BlockSpec((BLK, H), blk),
        out_shape=jax.ShapeDtypeStruct((N, H), jnp.float32),
    )(hs[0], hs[1], hs[2],
      p['Wih_f'], p['Whh_f'], p['b_f'][None, :],
      p['Wih_b'], p['Whh_b'], p['b_b'][None, :],
      p['att_W'].T, p['att_b'][None, :])


def _pool_body(h_ref, batch_ref, hls_ref,
               w1p_ref, b1p_ref, w2p_ref, b2p_ref,
               w1t_ref, b1t_ref, w2t_ref, b2t_ref,
               mw0_ref, mb0_ref, mw1_ref, mb1_ref, mw2_ref, mb2_ref,
               out_ref, nump, denp, numt, dent):
    i = pl.program_id(0)
    h = h_ref[...]
    bt = batch_ref[...]
    a = (bt == lax.broadcasted_iota(jnp.int32, (BLK, B), 1)).astype(jnp.float32)

    def gate(w1_ref, b1_ref, w2_ref, b2_ref):
        g1 = jnp.maximum(
            jnp.dot(h, w1_ref[...], preferred_element_type=jnp.float32) + b1_ref[...], 0.0)
        g = jnp.dot(g1, w2_ref[...], preferred_element_type=jnp.float32) + b2_ref[...]
        return jnp.exp(g)

    egp = gate(w1p_ref, b1p_ref, w2p_ref, b2p_ref)
    egt = gate(w1t_ref, b1t_ref, w2t_ref, b2t_ref)

    dn = (((0,), (0,)), ((), ()))

    @pl.when(i == 0)
    def _():
        nump[...] = jnp.zeros_like(nump)
        denp[...] = jnp.zeros_like(denp)
        numt[...] = jnp.zeros_like(numt)
        dent[...] = jnp.zeros_like(dent)

    nump[...] += lax.dot_general(a, egp * h, dn, preferred_element_type=jnp.float32)
    denp[...] += lax.dot_general(a, egp, dn, preferred_element_type=jnp.float32)
    numt[...] += lax.dot_general(a, egt * h, dn, preferred_element_type=jnp.float32)
    dent[...] += lax.dot_general(a, egt, dn, preferred_element_type=jnp.float32)

    @pl.when(i == NB - 1)
    def _():
        outp = nump[...] / jnp.maximum(denp[...], 1e-16)
        outt = numt[...] / jnp.maximum(dent[...], 1e-16)
        zcat = jnp.concatenate([outp, outt, hls_ref[...]], axis=1)
        z1 = jnp.maximum(
            jnp.dot(zcat, mw0_ref[...], preferred_element_type=jnp.float32) + mb0_ref[...], 0.0)
        z2 = jnp.maximum(
            jnp.dot(z1, mw1_ref[...], preferred_element_type=jnp.float32) + mb1_ref[...], 0.0)
        out_ref[...] = jnp.dot(z2, mw2_ref[...], preferred_element_type=jnp.float32) + mb2_ref[...]


def _pool(h, batch2d, hls_attr, gp, gt, mlps):
    blk = lambda i: (i, 0)
    full = lambda i: (0, 0)
    return pl.pallas_call(
        _pool_body,
        grid=(NB,),
        in_specs=[
            pl.BlockSpec((BLK, H), blk),
            pl.BlockSpec((BLK, 1), blk),
            pl.BlockSpec((B, 64), full),
            pl.BlockSpec((H, H), full),
            pl.BlockSpec((1, H), full),
            pl.BlockSpec((H, 1), full),
            pl.BlockSpec((1, 1), full),
            pl.BlockSpec((H, H), full),
            pl.BlockSpec((1, H), full),
            pl.BlockSpec((H, 1), full),
            pl.BlockSpec((1, 1), full),
            pl.BlockSpec((2 * H + 64, 64), full),
            pl.BlockSpec((1, 64), full),
            pl.BlockSpec((64, 64), full),
            pl.BlockSpec((1, 64), full),
            pl.BlockSpec((64, 1), full),
            pl.BlockSpec((1, 1), full),
        ],
        out_specs=pl.BlockSpec((B, 1), full),
        out_shape=jax.ShapeDtypeStruct((B, 1), jnp.float32),
        scratch_shapes=[
            pltpu.VMEM((B, H), jnp.float32),
            pltpu.VMEM((B, 1), jnp.float32),
            pltpu.VMEM((B, H), jnp.float32),
            pltpu.VMEM((B, 1), jnp.float32),
        ],
    )(h, batch2d, hls_attr,
      gp['W1'], gp['b1'][None, :], gp['W2'], gp['b2'][None, :],
      gt['W1'], gt['b1'][None, :], gt['W2'], gt['b2'][None, :],
      mlps[0]['W'], mlps[0]['b'][None, :],
      mlps[1]['W'], mlps[1]['b'][None, :],
      mlps[2]['W'], mlps[2]['b'][None, :])


def kernel(x, edge_index, batch, hls_attr, edge_attr, params):
    src = edge_index[0]
    dst = edge_index[1]
    h = x
    hs = []
    for p in params['convs']:
        w4 = jnp.concatenate([p['Wq'], p['Wk'], p['Wv'], p['Wskip']], axis=1)
        b4 = jnp.concatenate([p['bq'], p['bk'], p['bv'], p['bskip']])[None, :]
        qqe, kv, skip = _proj(h, w4, b4, p['We'].T)
        pa = _edge(qqe, kv, edge_attr, src, dst)
        h = _finalize(pa[:, :N], skip, p['We'])
        hs.append(h)
    jk = _lstm(hs, params['lstm'])
    return _pool(jk, batch[:, None], hls_attr,
                 params['glob_P'], params['glob_T'], params['mlps'])
